# baseline (speedup 1.0000x reference)
import functools
import math

import numpy as np
import jax
import jax.numpy as jnp
from jax import lax
from jax.experimental import pallas as pl
from jax.experimental.pallas import tpu as pltpu
from jax.experimental.pallas import tpu_sc as plsc

VOCAB = 1000000
D = 64
B = 4096
L = 200

NC = 2
NS = 16
NW = NC * NS
SEQ_PER_W = B // NW
NBUF = 4

SCALE = 2.0 * math.sqrt(D)


def _make_pe() -> np.ndarray:
    position = np.arange(0, L, dtype=np.float32)[:, None]
    div_even = np.power(10000.0, np.arange(0, D, 2, dtype=np.float32) / D)
    div_odd = np.power(10000.0, np.arange(1, D, 2, dtype=np.float32) / D)
    pe = np.zeros((L, D), dtype=np.float32)
    pe[:, 0::2] = np.sin(position * div_even)
    pe[:, 1::2] = np.cos(position * div_odd)
    return pe


_PE = _make_pe()


@functools.cache
def _build():
    mesh = plsc.VectorSubcoreMesh(
        core_axis_name="c", subcore_axis_name="s", num_cores=NC, num_subcores=NS
    )

    @functools.partial(
        pl.kernel,
        mesh=mesh,
        out_type=jax.ShapeDtypeStruct((B, L, D), jnp.float32),
        scratch_types=[
            [pltpu.VMEM((L,), jnp.int32) for _ in range(NBUF)],
            [pltpu.VMEM((L, D), jnp.float32) for _ in range(NBUF)],
            pltpu.VMEM((L, D), jnp.float32),
            [pltpu.SemaphoreType.DMA for _ in range(NBUF)],
            [pltpu.SemaphoreType.DMA for _ in range(NBUF)],
            [pltpu.SemaphoreType.DMA for _ in range(NBUF)],
        ],
        compiler_params=pltpu.CompilerParams(
            use_tc_tiling_on_sc=False, skip_device_barrier=True
        ),
    )
    def emb(x_hbm, table_hbm, pe_hbm, out_hbm, idx_v, rows_v, pe_v, isem, gsem, ssem):
        wid = lax.axis_index("s") * NC + lax.axis_index("c")
        base = wid * SEQ_PER_W
        pltpu.sync_copy(pe_hbm, pe_v)

        def idx_start(b, k):
            pltpu.async_copy(x_hbm.at[base + k], idx_v[b], isem[b])

        def idx_wait(b):
            pltpu.make_async_copy(x_hbm.at[base], idx_v[b], isem[b]).wait()

        def gather_start(b, k):
            pltpu.async_copy(table_hbm.at[idx_v[b]], rows_v[b], gsem[b])

        def gather_wait(b):
            pltpu.make_async_copy(
                table_hbm.at[idx_v[b]], rows_v[b], gsem[b]
            ).wait()

        def store_wait(b):
            pltpu.make_async_copy(rows_v[b], out_hbm.at[base], ssem[b]).wait()

        idx_start(0, 0)
        idx_start(1, 1)
        idx_wait(0)
        gather_start(0, 0)

        def body(i, carry):
            for j in range(NBUF):
                k = i * NBUF + j
                bn = (j + 1) % NBUF
                bi = (j + 2) % NBUF

                @pl.when(k + 2 < SEQ_PER_W)
                def _():
                    idx_start(bi, k + 2)

                @pl.when(k + 1 < SEQ_PER_W)
                def _():
                    idx_wait(bn)

                    @pl.when(k >= NBUF - 1)
                    def _():
                        store_wait(bn)

                    gather_start(bn, k + 1)

                gather_wait(j)

                def per_r(r4, c):
                    for ri in range(4):
                        r = r4 * 4 + ri
                        for kk in range(D // 16):
                            sl = pl.ds(16 * kk, 16)
                            rows_v[j][r, sl] = (
                                rows_v[j][r, sl] * SCALE + pe_v[r, sl]
                            )
                    return c

                lax.fori_loop(0, L // 4, per_r, 0)
                pltpu.async_copy(rows_v[j], out_hbm.at[base + k], ssem[j])
            return carry

        lax.fori_loop(0, SEQ_PER_W // NBUF, body, 0)

        for j in range(NBUF):
            store_wait(j)

    return emb


def kernel(x, table):
    return _build()(x, table, _PE)
